# baseline (speedup 1.0000x reference)
import jax
import jax.numpy as jnp
from jax import lax
from jax.experimental import pallas as pl
from jax.experimental.pallas import tpu as pltpu
from jax.experimental.pallas import tpu_sc as plsc

N = 10000
E = 160000
HID = 512
NPLANES = 4
LANES = 128
NC = 2
NS = 16
EPS = E // NS
CHUNK = 128
NCHUNKS = (EPS + CHUNK - 1) // CHUNK
EPAD = NCHUNKS * CHUNK
NP = N + 112
ZPS = NP // NS
ZR = 64
RB = 1000
GRID = N // RB
EPS32 = E // (NC * NS)
NCH32 = (EPS32 + CHUNK - 1) // CHUNK
EPAD32 = NCH32 * CHUNK

_MESH = plsc.VectorSubcoreMesh(core_axis_name="c", subcore_axis_name="s",
                               num_cores=NC, num_subcores=NS)


def _deg_body(idx_hbm, deg_hbm, idx2d, ones_v, zbuf, acc):
    c = lax.axis_index("c")
    s = lax.axis_index("s")
    pltpu.sync_copy(idx_hbm.at[c, s], idx2d)

    def fill1(i, _):
        ones_v[i] = jnp.full((16,), 1.0, jnp.float32)
        return 0
    lax.fori_loop(0, CHUNK, fill1, 0)

    def fill0(i, _):
        zbuf[i] = jnp.zeros((16,), jnp.float32)
        return 0
    lax.fori_loop(0, ZR, fill0, 0)

    base = s * ZPS
    def zloop(i, _):
        pltpu.sync_copy(zbuf, acc.at[pl.ds(base + i * ZR, ZR)])
        return 0
    lax.fori_loop(0, ZPS // ZR, zloop, 0)
    pltpu.sync_copy(zbuf.at[pl.ds(0, ZPS % ZR)],
                    acc.at[pl.ds(base + (ZPS // ZR) * ZR, ZPS % ZR)])
    plsc.subcore_barrier()

    def sloop(j, _):
        pltpu.sync_copy(ones_v, acc.at[idx2d.at[j]], add=True)
        return 0
    lax.fori_loop(0, NCHUNKS, sloop, 0)
    plsc.subcore_barrier()

    pltpu.sync_copy(acc.at[pl.ds(base, ZPS)], deg_hbm.at[c, pl.ds(base, ZPS)])


def _sc_degrees(degidx):
    return pl.kernel(
        _deg_body,
        out_type=jax.ShapeDtypeStruct((NC, NP, 16), jnp.float32),
        mesh=_MESH,
        scratch_types=[
            pltpu.VMEM((NCHUNKS, CHUNK), jnp.int32),
            pltpu.VMEM((CHUNK, 16), jnp.float32),
            pltpu.VMEM((ZR, 16), jnp.float32),
            pltpu.VMEM_SHARED((NP, 16), jnp.float32),
        ],
    )(degidx)


def _unpack_src(pidx, srcbuf, j, slot):
    for k in range(CHUNK // 16):
        v = pidx[j, pl.ds(k * 16, 16)]
        srcbuf[slot, pl.ds(k * 16, 16)] = lax.bitwise_and(v, 0xFFFF)


def _unpack_dst(pidx, dstbuf, j):
    for k in range(CHUNK // 16):
        v = pidx[j, pl.ds(k * 16, 16)]
        dstbuf[0, pl.ds(k * 16, 16)] = lax.shift_right_logical(v, 16)


def _agg_body(t_hbm, pidx_hbm, out_hbm,
              pidx, srcbuf, dstbuf, gbuf, acc, sem):
    c = lax.axis_index("c")
    s = lax.axis_index("s")
    base = s * ZPS
    for pi in range(2):
        g = c * 2 + pi
        pltpu.sync_copy(pidx_hbm.at[g, s], pidx)

        def fill0(i, _):
            gbuf[0, i // 8, pl.ds((i % 8) * 16, 16)] = \
                jnp.zeros((16,), jnp.float32)
            return 0
        lax.fori_loop(0, CHUNK * 8, fill0, 0)

        def zloop(i, _):
            pltpu.sync_copy(gbuf.at[0], acc.at[pl.ds(base + i * CHUNK, CHUNK)])
            return 0
        lax.fori_loop(0, ZPS // CHUNK, zloop, 0)
        pltpu.sync_copy(gbuf.at[0, pl.ds(0, ZPS % CHUNK)],
                        acc.at[pl.ds(base + (ZPS // CHUNK) * CHUNK,
                                     ZPS % CHUNK)])
        plsc.subcore_barrier()

        _unpack_src(pidx, srcbuf, 0, 0)
        pltpu.async_copy(t_hbm.at[srcbuf.at[0]], gbuf.at[0], sem.at[0])

        def sloop(j, _):
            slot = lax.rem(j, 2)
            nslot = lax.rem(j + 1, 2)

            @pl.when(j < NCHUNKS - 1)
            def _():
                _unpack_src(pidx, srcbuf, j + 1, nslot)
                pltpu.async_copy(t_hbm.at[srcbuf.at[nslot]], gbuf.at[nslot],
                                 sem.at[nslot])

            _unpack_dst(pidx, dstbuf, j)
            pltpu.make_async_copy(t_hbm.at[srcbuf.at[slot]], gbuf.at[slot],
                                  sem.at[slot]).wait()
            pltpu.sync_copy(gbuf.at[slot], acc.at[dstbuf.at[0]], add=True)
            return 0
        lax.fori_loop(0, NCHUNKS, sloop, 0)
        plsc.subcore_barrier()

        pltpu.sync_copy(acc.at[pl.ds(base, ZPS)],
                        out_hbm.at[g, pl.ds(base, ZPS)])
        plsc.subcore_barrier()


def _sc_aggregate(t2d, pidx):
    return pl.kernel(
        _agg_body,
        out_type=jax.ShapeDtypeStruct((NPLANES, NP, LANES), jnp.float32),
        mesh=_MESH,
        scratch_types=[
            pltpu.VMEM((NCHUNKS, CHUNK), jnp.int32),
            pltpu.VMEM((2, CHUNK), jnp.int32),
            pltpu.VMEM((1, CHUNK), jnp.int32),
            pltpu.VMEM((2, CHUNK, LANES), jnp.float32),
            pltpu.VMEM_SHARED((NP, LANES), jnp.float32),
            pltpu.SemaphoreType.DMA((2,)),
        ],
    )(t2d, pidx)


def _agg16_body(x_hbm, pidx_hbm, out_hbm,
                pidx, srcbuf, dstbuf, gbuf, acc, sem):
    c = lax.axis_index("c")
    s = lax.axis_index("s")
    w = c * NS + s
    pltpu.sync_copy(pidx_hbm.at[w], pidx)

    def fill0(i, _):
        gbuf[0, i // 8, pl.ds((i % 8) * 16, 16)] = \
            jnp.zeros((16,), jnp.float32)
        return 0
    lax.fori_loop(0, CHUNK * 8, fill0, 0)

    base = s * ZPS
    def zloop(i, _):
        pltpu.sync_copy(gbuf.at[0], acc.at[pl.ds(base + i * CHUNK, CHUNK)])
        return 0
    lax.fori_loop(0, ZPS // CHUNK, zloop, 0)
    pltpu.sync_copy(gbuf.at[0, pl.ds(0, ZPS % CHUNK)],
                    acc.at[pl.ds(base + (ZPS // CHUNK) * CHUNK,
                                 ZPS % CHUNK)])
    plsc.subcore_barrier()

    _unpack_src(pidx, srcbuf, 0, 0)
    pltpu.async_copy(x_hbm.at[srcbuf.at[0]], gbuf.at[0], sem.at[0])

    def sloop(j, _):
        slot = lax.rem(j, 2)
        nslot = lax.rem(j + 1, 2)

        @pl.when(j < NCH32 - 1)
        def _():
            _unpack_src(pidx, srcbuf, j + 1, nslot)
            pltpu.async_copy(x_hbm.at[srcbuf.at[nslot]], gbuf.at[nslot],
                             sem.at[nslot])

        _unpack_dst(pidx, dstbuf, j)
        pltpu.make_async_copy(x_hbm.at[srcbuf.at[slot]], gbuf.at[slot],
                              sem.at[slot]).wait()
        pltpu.sync_copy(gbuf.at[slot], acc.at[dstbuf.at[0]], add=True)
        return 0
    lax.fori_loop(0, NCH32, sloop, 0)
    plsc.subcore_barrier()

    pltpu.sync_copy(acc.at[pl.ds(base, ZPS)], out_hbm.at[c, pl.ds(base, ZPS)])


def _sc_aggregate16(x1p, pidx32):
    return pl.kernel(
        _agg16_body,
        out_type=jax.ShapeDtypeStruct((NC, NP, LANES), jnp.float32),
        mesh=_MESH,
        scratch_types=[
            pltpu.VMEM((NCH32, CHUNK), jnp.int32),
            pltpu.VMEM((2, CHUNK), jnp.int32),
            pltpu.VMEM((1, CHUNK), jnp.int32),
            pltpu.VMEM((2, CHUNK, LANES), jnp.float32),
            pltpu.VMEM_SHARED((NP, LANES), jnp.float32),
            pltpu.SemaphoreType.DMA((2,)),
        ],
    )(x1p, pidx32)


def _prep_body(deg_ref, x1_ref, ns_ref, nd_ref):
    ind = deg_ref[0, :, 0:1]
    outd = deg_ref[1, :, 0:1]
    h1 = ind
    h2 = (ind > 3.0).astype(jnp.float32)
    h3 = 3.0 / ind
    h4 = (ind > 4.0).astype(jnp.float32)
    ns = lax.rsqrt(jnp.where(outd > 0.0, outd, 1.0))
    nd = lax.rsqrt(jnp.where(ind > 0.0, ind, 1.0))
    ns_ref[...] = ns
    nd_ref[...] = nd
    x1_ref[...] = jnp.concatenate(
        [h1, h2, h3, h4, jnp.zeros((RB, LANES - 4), jnp.float32)],
        axis=1) * ns


def _tc_prep(deg):
    return pl.pallas_call(
        _prep_body,
        grid=(GRID,),
        in_specs=[
            pl.BlockSpec((NC, RB, 16), lambda r: (0, r, 0)),
        ],
        out_specs=[
            pl.BlockSpec((RB, LANES), lambda r: (r, 0)),
            pl.BlockSpec((RB, 1), lambda r: (r, 0)),
            pl.BlockSpec((RB, 1), lambda r: (r, 0)),
        ],
        out_shape=[
            jax.ShapeDtypeStruct((N, LANES), jnp.float32),
            jax.ShapeDtypeStruct((N, 1), jnp.float32),
            jax.ShapeDtypeStruct((N, 1), jnp.float32),
        ],
    )(deg)


def _mid1_body(agg_ref, nd_ref, ns_ref, w1_ref, b_ref, w_ref, t_ref):
    nd = nd_ref[...]
    ns = ns_ref[...]
    a = jnp.dot((agg_ref[0] + agg_ref[1])[:, :16], w1_ref[...],
                preferred_element_type=jnp.float32)
    y = jnp.maximum(a * nd + b_ref[...], 0.0) * ns
    for g in range(NPLANES):
        t_ref[g] = jnp.dot(y, w_ref[:, g * LANES:(g + 1) * LANES],
                           preferred_element_type=jnp.float32)


def _tc_mid1(agg16, nd, ns, w1p, b, w):
    return pl.pallas_call(
        _mid1_body,
        grid=(GRID,),
        in_specs=[
            pl.BlockSpec((NC, RB, LANES), lambda r: (0, r, 0)),
            pl.BlockSpec((RB, 1), lambda r: (r, 0)),
            pl.BlockSpec((RB, 1), lambda r: (r, 0)),
            pl.BlockSpec((16, HID), lambda r: (0, 0)),
            pl.BlockSpec((1, HID), lambda r: (0, 0)),
            pl.BlockSpec((HID, HID), lambda r: (0, 0)),
        ],
        out_specs=pl.BlockSpec((NPLANES, RB, LANES), lambda r: (0, r, 0)),
        out_shape=jax.ShapeDtypeStruct((NPLANES, N, LANES), jnp.float32),
    )(agg16, nd, ns, w1p, b, w)


def _mid_body(agg_ref, nd_ref, ns_ref, b_ref, w_ref, t_ref):
    nd = nd_ref[...]
    ns = ns_ref[...]
    ys = []
    for g in range(NPLANES):
        a = agg_ref[g]
        ys.append(jnp.maximum(a * nd + b_ref[0:1, g * LANES:(g + 1) * LANES],
                              0.0) * ns)
    y = jnp.concatenate(ys, axis=1)
    for g in range(NPLANES):
        t_ref[g] = jnp.dot(y, w_ref[:, g * LANES:(g + 1) * LANES],
                           preferred_element_type=jnp.float32)


def _tc_mid(agg, nd, ns, b, w):
    return pl.pallas_call(
        _mid_body,
        grid=(GRID,),
        in_specs=[
            pl.BlockSpec((NPLANES, RB, LANES), lambda r: (0, r, 0)),
            pl.BlockSpec((RB, 1), lambda r: (r, 0)),
            pl.BlockSpec((RB, 1), lambda r: (r, 0)),
            pl.BlockSpec((1, HID), lambda r: (0, 0)),
            pl.BlockSpec((HID, HID), lambda r: (0, 0)),
        ],
        out_specs=pl.BlockSpec((NPLANES, RB, LANES), lambda r: (0, r, 0)),
        out_shape=jax.ShapeDtypeStruct((NPLANES, N, LANES), jnp.float32),
    )(agg, nd, ns, b, w)


def _fin_body(agg_ref, nd_ref, b_ref, h_ref):
    nd = nd_ref[...]
    hs = [jnp.maximum(agg_ref[g] * nd + b_ref[0:1, g * LANES:(g + 1) * LANES],
                      0.0) for g in range(NPLANES)]
    h_ref[...] = jnp.concatenate(hs, axis=1)


def _tc_final(agg, nd, b):
    return pl.pallas_call(
        _fin_body,
        grid=(GRID,),
        in_specs=[
            pl.BlockSpec((NPLANES, RB, LANES), lambda r: (0, r, 0)),
            pl.BlockSpec((RB, 1), lambda r: (r, 0)),
            pl.BlockSpec((1, HID), lambda r: (0, 0)),
        ],
        out_specs=pl.BlockSpec((RB, HID), lambda r: (r, 0)),
        out_shape=jax.ShapeDtypeStruct((N, HID), jnp.float32),
    )(agg, nd, b)


def _ro_body(h_ref, lw1_ref, lb1_ref, lw2_ref, lb2_ref, pred_ref, ge_ref):
    sm = jnp.sum(h_ref[...], axis=0, keepdims=True) / jnp.float32(N)
    ge_ref[...] = sm
    e = jnp.maximum(
        jnp.dot(sm, lw1_ref[...], preferred_element_type=jnp.float32)
        + lb1_ref[...], 0.0)
    p = (jnp.dot(e, lw2_ref[...], preferred_element_type=jnp.float32)
         + lb2_ref[...])
    pred_ref[...] = jax.nn.sigmoid(p)


def _tc_readout(h_nodes, lw1, lb1, lw2, lb2):
    return pl.pallas_call(
        _ro_body,
        out_shape=[
            jax.ShapeDtypeStruct((1, 1), jnp.float32),
            jax.ShapeDtypeStruct((1, HID), jnp.float32),
        ],
    )(h_nodes, lw1, lb1, lw2, lb2)


def kernel(edge_index, W1, b1, W2, b2, W3, b3, W4, b4, W5, b5,
           lw1, lb1, lw2, lb2):
    src = edge_index[0]
    dst = edge_index[1]
    padN = jnp.full((NS, EPAD - EPS), N, jnp.int32)
    pad0 = jnp.zeros((NS, EPAD - EPS), jnp.int32)
    dst_sub = jnp.concatenate([dst.reshape(NS, EPS), padN], axis=1)
    src_sub = jnp.concatenate([src.reshape(NS, EPS), pad0], axis=1)
    dstidx = dst_sub.reshape(NS, NCHUNKS, CHUNK)
    pidx = ((dst_sub << 16) | src_sub)[None] + \
        (jnp.arange(NPLANES, dtype=jnp.int32) * N)[:, None, None]
    pidx = pidx.reshape(NPLANES, NS, NCHUNKS, CHUNK)
    src_subN = jnp.concatenate([src.reshape(NS, EPS), padN], axis=1)
    degidx = jnp.stack([dstidx, src_subN.reshape(NS, NCHUNKS, CHUNK)])
    padN32 = jnp.full((NC * NS, EPAD32 - EPS32), N, jnp.int32)
    pad032 = jnp.zeros((NC * NS, EPAD32 - EPS32), jnp.int32)
    dst32 = jnp.concatenate([dst.reshape(NC * NS, EPS32), padN32], axis=1)
    src32 = jnp.concatenate([src.reshape(NC * NS, EPS32), pad032], axis=1)
    pidx32 = ((dst32 << 16) | src32).reshape(NC * NS, NCH32, CHUNK)

    deg = _sc_degrees(degidx)
    w1p = jnp.pad(W1, ((0, 12), (0, 0)))
    x1, ns, nd = _tc_prep(deg)
    agg16 = _sc_aggregate16(x1, pidx32)
    t = _tc_mid1(agg16, nd, ns, w1p, b1.reshape(1, HID), W2)
    agg = _sc_aggregate(t.reshape(NPLANES * N, LANES), pidx)
    for bprev, W in ((b2, W3), (b3, W4), (b4, W5)):
        t = _tc_mid(agg, nd, ns, bprev.reshape(1, HID), W)
        agg = _sc_aggregate(t.reshape(NPLANES * N, LANES), pidx)
    h_nodes = _tc_final(agg, nd, b5.reshape(1, HID))
    pred, graph_emb = _tc_readout(h_nodes, lw1, lb1.reshape(1, 256),
                                  lw2, lb2.reshape(1, 1))
    return (pred, graph_emb, h_nodes)


# speedup vs baseline: 1.0037x; 1.0037x over previous
import jax
import jax.numpy as jnp
from jax import lax
from jax.experimental import pallas as pl
from jax.experimental.pallas import tpu as pltpu
from jax.experimental.pallas import tpu_sc as plsc

N = 10000
E = 160000
HID = 512
NPLANES = 4
LANES = 128
NC = 2
NS = 16
EPS = E // NS
CHUNK = 128
NCHUNKS = (EPS + CHUNK - 1) // CHUNK
EPAD = NCHUNKS * CHUNK
NP = N + 112
ZPS = NP // NS
ZR = 64
RB = 1000
GRID = N // RB
EPS32 = E // (NC * NS)
NCH32 = (EPS32 + CHUNK - 1) // CHUNK
EPAD32 = NCH32 * CHUNK

_MESH = plsc.VectorSubcoreMesh(core_axis_name="c", subcore_axis_name="s",
                               num_cores=NC, num_subcores=NS)


def _deg_body(idx_hbm, deg_hbm, idx2d, ones_v, zbuf, acc):
    c = lax.axis_index("c")
    s = lax.axis_index("s")
    pltpu.sync_copy(idx_hbm.at[c, s], idx2d)

    def fill1(i, _):
        ones_v[i] = jnp.full((16,), 1.0, jnp.float32)
        return 0
    lax.fori_loop(0, CHUNK, fill1, 0)

    def fill0(i, _):
        zbuf[i] = jnp.zeros((16,), jnp.float32)
        return 0
    lax.fori_loop(0, ZR, fill0, 0)

    base = s * ZPS
    def zloop(i, _):
        pltpu.sync_copy(zbuf, acc.at[pl.ds(base + i * ZR, ZR)])
        return 0
    lax.fori_loop(0, ZPS // ZR, zloop, 0)
    pltpu.sync_copy(zbuf.at[pl.ds(0, ZPS % ZR)],
                    acc.at[pl.ds(base + (ZPS // ZR) * ZR, ZPS % ZR)])
    plsc.subcore_barrier()

    def sloop(j, _):
        pltpu.sync_copy(ones_v, acc.at[idx2d.at[j]], add=True)
        return 0
    lax.fori_loop(0, NCHUNKS, sloop, 0)
    plsc.subcore_barrier()

    pltpu.sync_copy(acc.at[pl.ds(base, ZPS)], deg_hbm.at[c, pl.ds(base, ZPS)])


def _sc_degrees(degidx):
    return pl.kernel(
        _deg_body,
        out_type=jax.ShapeDtypeStruct((NC, NP, 16), jnp.float32),
        mesh=_MESH,
        scratch_types=[
            pltpu.VMEM((NCHUNKS, CHUNK), jnp.int32),
            pltpu.VMEM((CHUNK, 16), jnp.float32),
            pltpu.VMEM((ZR, 16), jnp.float32),
            pltpu.VMEM_SHARED((NP, 16), jnp.float32),
        ],
    )(degidx)


def _unpack_src(pidx, srcbuf, j, slot):
    for k in range(CHUNK // 16):
        v = pidx[j, pl.ds(k * 16, 16)]
        srcbuf[slot, pl.ds(k * 16, 16)] = lax.bitwise_and(v, 0xFFFF)


def _unpack_dst(pidx, dstbuf, j):
    for k in range(CHUNK // 16):
        v = pidx[j, pl.ds(k * 16, 16)]
        dstbuf[0, pl.ds(k * 16, 16)] = lax.shift_right_logical(v, 16)


def _agg_body(t_hbm, pidx_hbm, out_hbm,
              pidx, srcbuf, dstbuf, gbuf, acc, sem):
    c = lax.axis_index("c")
    s = lax.axis_index("s")
    base = s * ZPS
    for pi in range(2):
        g = c * 2 + pi
        pltpu.sync_copy(pidx_hbm.at[g, s], pidx)

        def fill0(i, _):
            gbuf[0, i // 8, pl.ds((i % 8) * 16, 16)] = \
                jnp.zeros((16,), jnp.float32)
            return 0
        lax.fori_loop(0, CHUNK * 8, fill0, 0)

        def zloop(i, _):
            pltpu.sync_copy(gbuf.at[0], acc.at[pl.ds(base + i * CHUNK, CHUNK)])
            return 0
        lax.fori_loop(0, ZPS // CHUNK, zloop, 0)
        pltpu.sync_copy(gbuf.at[0, pl.ds(0, ZPS % CHUNK)],
                        acc.at[pl.ds(base + (ZPS // CHUNK) * CHUNK,
                                     ZPS % CHUNK)])
        plsc.subcore_barrier()

        _unpack_src(pidx, srcbuf, 0, 0)
        pltpu.async_copy(t_hbm.at[srcbuf.at[0]], gbuf.at[0], sem.at[0])

        def sloop(j, _):
            slot = lax.rem(j, 2)
            nslot = lax.rem(j + 1, 2)

            @pl.when(j < NCHUNKS - 1)
            def _():
                _unpack_src(pidx, srcbuf, j + 1, nslot)
                pltpu.async_copy(t_hbm.at[srcbuf.at[nslot]], gbuf.at[nslot],
                                 sem.at[nslot])

            _unpack_dst(pidx, dstbuf, j)
            pltpu.make_async_copy(t_hbm.at[srcbuf.at[slot]], gbuf.at[slot],
                                  sem.at[slot]).wait()
            pltpu.sync_copy(gbuf.at[slot], acc.at[dstbuf.at[0]], add=True)
            return 0
        lax.fori_loop(0, NCHUNKS, sloop, 0)
        plsc.subcore_barrier()

        pltpu.sync_copy(acc.at[pl.ds(base, ZPS)],
                        out_hbm.at[g, pl.ds(base, ZPS)])
        plsc.subcore_barrier()


def _sc_aggregate(t2d, pidx):
    return pl.kernel(
        _agg_body,
        out_type=jax.ShapeDtypeStruct((NPLANES, NP, LANES), jnp.float32),
        mesh=_MESH,
        scratch_types=[
            pltpu.VMEM((NCHUNKS, CHUNK), jnp.int32),
            pltpu.VMEM((2, CHUNK), jnp.int32),
            pltpu.VMEM((1, CHUNK), jnp.int32),
            pltpu.VMEM((2, CHUNK, LANES), jnp.float32),
            pltpu.VMEM_SHARED((NP, LANES), jnp.float32),
            pltpu.SemaphoreType.DMA((2,)),
        ],
    )(t2d, pidx)


def _agg16_body(x_hbm, pidx_hbm, out_hbm,
                pidx, srcbuf, dstbuf, gbuf, acc, sem):
    c = lax.axis_index("c")
    s = lax.axis_index("s")
    w = c * NS + s
    pltpu.sync_copy(pidx_hbm.at[w], pidx)

    def fill0(i, _):
        gbuf[0, i // 8, pl.ds((i % 8) * 16, 16)] = \
            jnp.zeros((16,), jnp.float32)
        return 0
    lax.fori_loop(0, CHUNK * 8, fill0, 0)

    base = s * ZPS
    def zloop(i, _):
        pltpu.sync_copy(gbuf.at[0], acc.at[pl.ds(base + i * CHUNK, CHUNK)])
        return 0
    lax.fori_loop(0, ZPS // CHUNK, zloop, 0)
    pltpu.sync_copy(gbuf.at[0, pl.ds(0, ZPS % CHUNK)],
                    acc.at[pl.ds(base + (ZPS // CHUNK) * CHUNK,
                                 ZPS % CHUNK)])
    plsc.subcore_barrier()

    _unpack_src(pidx, srcbuf, 0, 0)
    pltpu.async_copy(x_hbm.at[srcbuf.at[0]], gbuf.at[0], sem.at[0])

    def sloop(j, _):
        slot = lax.rem(j, 2)
        nslot = lax.rem(j + 1, 2)

        @pl.when(j < NCH32 - 1)
        def _():
            _unpack_src(pidx, srcbuf, j + 1, nslot)
            pltpu.async_copy(x_hbm.at[srcbuf.at[nslot]], gbuf.at[nslot],
                             sem.at[nslot])

        _unpack_dst(pidx, dstbuf, j)
        pltpu.make_async_copy(x_hbm.at[srcbuf.at[slot]], gbuf.at[slot],
                              sem.at[slot]).wait()
        pltpu.sync_copy(gbuf.at[slot], acc.at[dstbuf.at[0]], add=True)
        return 0
    lax.fori_loop(0, NCH32, sloop, 0)
    plsc.subcore_barrier()

    pltpu.sync_copy(acc.at[pl.ds(base, ZPS)], out_hbm.at[c, pl.ds(base, ZPS)])


def _sc_aggregate16(x1p, pidx32):
    return pl.kernel(
        _agg16_body,
        out_type=jax.ShapeDtypeStruct((NC, NP, LANES), jnp.float32),
        mesh=_MESH,
        scratch_types=[
            pltpu.VMEM((NCH32, CHUNK), jnp.int32),
            pltpu.VMEM((2, CHUNK), jnp.int32),
            pltpu.VMEM((1, CHUNK), jnp.int32),
            pltpu.VMEM((2, CHUNK, LANES), jnp.float32),
            pltpu.VMEM_SHARED((NP, LANES), jnp.float32),
            pltpu.SemaphoreType.DMA((2,)),
        ],
    )(x1p, pidx32)


def _prep_body(deg_ref, x1_ref, ns_ref, nd_ref):
    ind = deg_ref[0, :, 0:1]
    outd = deg_ref[1, :, 0:1]
    h1 = ind
    h2 = (ind > 3.0).astype(jnp.float32)
    h3 = 3.0 / ind
    h4 = (ind > 4.0).astype(jnp.float32)
    ns = lax.rsqrt(jnp.where(outd > 0.0, outd, 1.0))
    nd = lax.rsqrt(jnp.where(ind > 0.0, ind, 1.0))
    ns_ref[...] = ns
    nd_ref[...] = nd
    x1_ref[...] = jnp.concatenate(
        [h1, h2, h3, h4, jnp.zeros((RB, LANES - 4), jnp.float32)],
        axis=1) * ns


def _tc_prep(deg):
    return pl.pallas_call(
        _prep_body,
        grid=(GRID,),
        in_specs=[
            pl.BlockSpec((NC, RB, 16), lambda r: (0, r, 0)),
        ],
        out_specs=[
            pl.BlockSpec((RB, LANES), lambda r: (r, 0)),
            pl.BlockSpec((RB, 1), lambda r: (r, 0)),
            pl.BlockSpec((RB, 1), lambda r: (r, 0)),
        ],
        out_shape=[
            jax.ShapeDtypeStruct((N, LANES), jnp.float32),
            jax.ShapeDtypeStruct((N, 1), jnp.float32),
            jax.ShapeDtypeStruct((N, 1), jnp.float32),
        ],
    )(deg)


def _mid1_body(agg_ref, nd_ref, ns_ref, w1_ref, b_ref, w_ref, t_ref):
    nd = nd_ref[...]
    ns = ns_ref[...]
    a = jnp.dot((agg_ref[0] + agg_ref[1])[:, :16], w1_ref[...],
                preferred_element_type=jnp.float32)
    y = jnp.maximum(a * nd + b_ref[...], 0.0) * ns
    for g in range(NPLANES):
        t_ref[g] = jnp.dot(y, w_ref[:, g * LANES:(g + 1) * LANES],
                           preferred_element_type=jnp.float32)


def _tc_mid1(agg16, nd, ns, w1p, b, w):
    return pl.pallas_call(
        _mid1_body,
        grid=(GRID,),
        in_specs=[
            pl.BlockSpec((NC, RB, LANES), lambda r: (0, r, 0)),
            pl.BlockSpec((RB, 1), lambda r: (r, 0)),
            pl.BlockSpec((RB, 1), lambda r: (r, 0)),
            pl.BlockSpec((16, HID), lambda r: (0, 0)),
            pl.BlockSpec((1, HID), lambda r: (0, 0)),
            pl.BlockSpec((HID, HID), lambda r: (0, 0)),
        ],
        out_specs=pl.BlockSpec((NPLANES, RB, LANES), lambda r: (0, r, 0)),
        out_shape=jax.ShapeDtypeStruct((NPLANES, N, LANES), jnp.float32),
    )(agg16, nd, ns, w1p, b, w)


def _mid_body(agg_ref, nd_ref, ns_ref, b_ref, w_ref, t_ref):
    nd = nd_ref[...]
    ns = ns_ref[...]
    ys = []
    for g in range(NPLANES):
        a = agg_ref[g]
        ys.append(jnp.maximum(a * nd + b_ref[0:1, g * LANES:(g + 1) * LANES],
                              0.0) * ns)
    y = jnp.concatenate(ys, axis=1)
    for g in range(NPLANES):
        t_ref[g] = jnp.dot(y, w_ref[:, g * LANES:(g + 1) * LANES],
                           preferred_element_type=jnp.float32)


def _tc_mid(agg, nd, ns, b, w):
    return pl.pallas_call(
        _mid_body,
        grid=(GRID,),
        in_specs=[
            pl.BlockSpec((NPLANES, RB, LANES), lambda r: (0, r, 0)),
            pl.BlockSpec((RB, 1), lambda r: (r, 0)),
            pl.BlockSpec((RB, 1), lambda r: (r, 0)),
            pl.BlockSpec((1, HID), lambda r: (0, 0)),
            pl.BlockSpec((HID, HID), lambda r: (0, 0)),
        ],
        out_specs=pl.BlockSpec((NPLANES, RB, LANES), lambda r: (0, r, 0)),
        out_shape=jax.ShapeDtypeStruct((NPLANES, N, LANES), jnp.float32),
    )(agg, nd, ns, b, w)


def _fin_body(agg_ref, nd_ref, b_ref, lw1_ref, lb1_ref, lw2_ref, lb2_ref,
              h_ref, pred_ref, ge_ref, acc_ref):
    r = pl.program_id(0)
    nd = nd_ref[...]
    hs = [jnp.maximum(agg_ref[g] * nd + b_ref[0:1, g * LANES:(g + 1) * LANES],
                      0.0) for g in range(NPLANES)]
    h = jnp.concatenate(hs, axis=1)
    h_ref[...] = h
    part = jnp.sum(h.reshape(RB // 8, 8, HID), axis=0)

    @pl.when(r == 0)
    def _():
        acc_ref[...] = part

    @pl.when(r > 0)
    def _():
        acc_ref[...] = acc_ref[...] + part

    @pl.when(r == GRID - 1)
    def _():
        sm = jnp.sum(acc_ref[...], axis=0, keepdims=True) / jnp.float32(N)
        ge_ref[...] = sm
        e = jnp.maximum(
            jnp.dot(sm, lw1_ref[...], preferred_element_type=jnp.float32)
            + lb1_ref[...], 0.0)
        p = (jnp.dot(e, lw2_ref[...], preferred_element_type=jnp.float32)
             + lb2_ref[...])
        pred_ref[...] = jax.nn.sigmoid(p)


def _tc_final(agg, nd, b, lw1, lb1, lw2, lb2):
    return pl.pallas_call(
        _fin_body,
        grid=(GRID,),
        in_specs=[
            pl.BlockSpec((NPLANES, RB, LANES), lambda r: (0, r, 0)),
            pl.BlockSpec((RB, 1), lambda r: (r, 0)),
            pl.BlockSpec((1, HID), lambda r: (0, 0)),
            pl.BlockSpec((HID, 256), lambda r: (0, 0)),
            pl.BlockSpec((1, 256), lambda r: (0, 0)),
            pl.BlockSpec((256, 1), lambda r: (0, 0)),
            pl.BlockSpec((1, 1), lambda r: (0, 0)),
        ],
        out_specs=[
            pl.BlockSpec((RB, HID), lambda r: (r, 0)),
            pl.BlockSpec((1, 1), lambda r: (0, 0)),
            pl.BlockSpec((1, HID), lambda r: (0, 0)),
        ],
        out_shape=[
            jax.ShapeDtypeStruct((N, HID), jnp.float32),
            jax.ShapeDtypeStruct((1, 1), jnp.float32),
            jax.ShapeDtypeStruct((1, HID), jnp.float32),
        ],
        scratch_shapes=[pltpu.VMEM((8, HID), jnp.float32)],
    )(agg, nd, b, lw1, lb1, lw2, lb2)


def kernel(edge_index, W1, b1, W2, b2, W3, b3, W4, b4, W5, b5,
           lw1, lb1, lw2, lb2):
    src = edge_index[0]
    dst = edge_index[1]
    padN = jnp.full((NS, EPAD - EPS), N, jnp.int32)
    pad0 = jnp.zeros((NS, EPAD - EPS), jnp.int32)
    dst_sub = jnp.concatenate([dst.reshape(NS, EPS), padN], axis=1)
    src_sub = jnp.concatenate([src.reshape(NS, EPS), pad0], axis=1)
    dstidx = dst_sub.reshape(NS, NCHUNKS, CHUNK)
    pidx = ((dst_sub << 16) | src_sub)[None] + \
        (jnp.arange(NPLANES, dtype=jnp.int32) * N)[:, None, None]
    pidx = pidx.reshape(NPLANES, NS, NCHUNKS, CHUNK)
    src_subN = jnp.concatenate([src.reshape(NS, EPS), padN], axis=1)
    degidx = jnp.stack([dstidx, src_subN.reshape(NS, NCHUNKS, CHUNK)])
    padN32 = jnp.full((NC * NS, EPAD32 - EPS32), N, jnp.int32)
    pad032 = jnp.zeros((NC * NS, EPAD32 - EPS32), jnp.int32)
    dst32 = jnp.concatenate([dst.reshape(NC * NS, EPS32), padN32], axis=1)
    src32 = jnp.concatenate([src.reshape(NC * NS, EPS32), pad032], axis=1)
    pidx32 = ((dst32 << 16) | src32).reshape(NC * NS, NCH32, CHUNK)

    deg = _sc_degrees(degidx)
    w1p = jnp.pad(W1, ((0, 12), (0, 0)))
    x1, ns, nd = _tc_prep(deg)
    agg16 = _sc_aggregate16(x1, pidx32)
    t = _tc_mid1(agg16, nd, ns, w1p, b1.reshape(1, HID), W2)
    agg = _sc_aggregate(t.reshape(NPLANES * N, LANES), pidx)
    for bprev, W in ((b2, W3), (b3, W4), (b4, W5)):
        t = _tc_mid(agg, nd, ns, bprev.reshape(1, HID), W)
        agg = _sc_aggregate(t.reshape(NPLANES * N, LANES), pidx)
    h_nodes, pred, graph_emb = _tc_final(
        agg, nd, b5.reshape(1, HID), lw1, lb1.reshape(1, 256),
        lw2, lb2.reshape(1, 1))
    return (pred, graph_emb, h_nodes)


# speedup vs baseline: 1.0347x; 1.0309x over previous
import jax
import jax.numpy as jnp
from jax import lax
from jax.experimental import pallas as pl
from jax.experimental.pallas import tpu as pltpu
from jax.experimental.pallas import tpu_sc as plsc

N = 10000
E = 160000
HID = 512
NPLANES = 4
LANES = 128
NC = 2
NS = 16
EPS = E // NS
CHUNK = 128
NCHUNKS = (EPS + CHUNK - 1) // CHUNK
EPAD = NCHUNKS * CHUNK
NP = N + 112
ZPS = NP // NS
ZR = 64
RB = 1000
GRID = N // RB
EPS32 = E // (NC * NS)
NCH32 = (EPS32 + CHUNK - 1) // CHUNK
EPAD32 = NCH32 * CHUNK

_MESH = plsc.VectorSubcoreMesh(core_axis_name="c", subcore_axis_name="s",
                               num_cores=NC, num_subcores=NS)


def _deg_body(idx_hbm, deg_hbm, idx2d, ones_v, zbuf, acc):
    c = lax.axis_index("c")
    s = lax.axis_index("s")
    pltpu.sync_copy(idx_hbm.at[c, s], idx2d)

    def fill1(i, _):
        ones_v[i] = jnp.full((16,), 1.0, jnp.float32)
        return 0
    lax.fori_loop(0, CHUNK, fill1, 0)

    def fill0(i, _):
        zbuf[i] = jnp.zeros((16,), jnp.float32)
        return 0
    lax.fori_loop(0, ZR, fill0, 0)

    base = s * ZPS
    def zloop(i, _):
        pltpu.sync_copy(zbuf, acc.at[pl.ds(base + i * ZR, ZR)])
        return 0
    lax.fori_loop(0, ZPS // ZR, zloop, 0)
    pltpu.sync_copy(zbuf.at[pl.ds(0, ZPS % ZR)],
                    acc.at[pl.ds(base + (ZPS // ZR) * ZR, ZPS % ZR)])
    plsc.subcore_barrier()

    def sloop(j, _):
        pltpu.sync_copy(ones_v, acc.at[idx2d.at[j]], add=True)
        return 0
    lax.fori_loop(0, NCHUNKS, sloop, 0)
    plsc.subcore_barrier()

    pltpu.sync_copy(acc.at[pl.ds(base, ZPS)], deg_hbm.at[c, pl.ds(base, ZPS)])


def _sc_degrees(degidx):
    return pl.kernel(
        _deg_body,
        out_type=jax.ShapeDtypeStruct((NC, NP, 16), jnp.float32),
        mesh=_MESH,
        scratch_types=[
            pltpu.VMEM((NCHUNKS, CHUNK), jnp.int32),
            pltpu.VMEM((CHUNK, 16), jnp.float32),
            pltpu.VMEM((ZR, 16), jnp.float32),
            pltpu.VMEM_SHARED((NP, 16), jnp.float32),
        ],
    )(degidx)


def _unpack_src(pidx, srcbuf, j, slot):
    for k in range(CHUNK // 16):
        v = pidx[j, pl.ds(k * 16, 16)]
        srcbuf[slot, pl.ds(k * 16, 16)] = lax.bitwise_and(v, 0xFFFF)


def _unpack_dst(pidx, dstbuf, j):
    for k in range(CHUNK // 16):
        v = pidx[j, pl.ds(k * 16, 16)]
        dstbuf[0, pl.ds(k * 16, 16)] = lax.shift_right_logical(v, 16)


def _agg_body(t_hbm, pidx_hbm, out_hbm,
              pidx, srcbuf, dstbuf, gbuf, zbuf, acc, sem):
    c = lax.axis_index("c")
    s = lax.axis_index("s")
    base = s * ZPS

    def fill0(i, _):
        zbuf[i // 8, pl.ds((i % 8) * 16, 16)] = jnp.zeros((16,), jnp.float32)
        return 0
    lax.fori_loop(0, 40 * 8, fill0, 0)

    for pi in range(2):
        g = c * 2 + pi
        pltpu.sync_copy(pidx_hbm.at[g, s], pidx)

        def zloop(i, _):
            pltpu.sync_copy(zbuf, acc.at[pl.ds(base + i * 40, 40)])
            return 0
        lax.fori_loop(0, ZPS // 40, zloop, 0)
        pltpu.sync_copy(zbuf.at[pl.ds(0, ZPS % 40)],
                        acc.at[pl.ds(base + (ZPS // 40) * 40, ZPS % 40)])
        plsc.subcore_barrier()

        _unpack_src(pidx, srcbuf, 0, 0)
        pltpu.async_copy(t_hbm.at[srcbuf.at[0]], gbuf.at[0], sem.at[0])

        def sloop(j, _):
            slot = lax.rem(j, 2)
            nslot = lax.rem(j + 1, 2)

            @pl.when(j < NCHUNKS - 1)
            def _():
                _unpack_src(pidx, srcbuf, j + 1, nslot)
                pltpu.async_copy(t_hbm.at[srcbuf.at[nslot]], gbuf.at[nslot],
                                 sem.at[nslot])

            _unpack_dst(pidx, dstbuf, j)
            pltpu.make_async_copy(t_hbm.at[srcbuf.at[slot]], gbuf.at[slot],
                                  sem.at[slot]).wait()
            pltpu.sync_copy(gbuf.at[slot], acc.at[dstbuf.at[0]], add=True)
            return 0
        lax.fori_loop(0, NCHUNKS, sloop, 0)
        plsc.subcore_barrier()

        pltpu.sync_copy(acc.at[pl.ds(base, ZPS)],
                        out_hbm.at[g, pl.ds(base, ZPS)])
        plsc.subcore_barrier()


def _sc_aggregate(t2d, pidx):
    return pl.kernel(
        _agg_body,
        out_type=jax.ShapeDtypeStruct((NPLANES, NP, LANES), jnp.float32),
        mesh=_MESH,
        scratch_types=[
            pltpu.VMEM((NCHUNKS, CHUNK), jnp.int32),
            pltpu.VMEM((2, CHUNK), jnp.int32),
            pltpu.VMEM((1, CHUNK), jnp.int32),
            pltpu.VMEM((2, CHUNK, LANES), jnp.float32),
            pltpu.VMEM((40, LANES), jnp.float32),
            pltpu.VMEM_SHARED((NP, LANES), jnp.float32),
            pltpu.SemaphoreType.DMA((2,)),
        ],
    )(t2d, pidx)


def _agg16_body(x_hbm, pidx_hbm, out_hbm,
                pidx, srcbuf, dstbuf, gbuf, zbuf, acc, sem):
    c = lax.axis_index("c")
    s = lax.axis_index("s")
    w = c * NS + s
    pltpu.sync_copy(pidx_hbm.at[w], pidx)

    def fill0(i, _):
        zbuf[i // 8, pl.ds((i % 8) * 16, 16)] = jnp.zeros((16,), jnp.float32)
        return 0
    lax.fori_loop(0, 40 * 8, fill0, 0)

    base = s * ZPS
    def zloop(i, _):
        pltpu.sync_copy(zbuf, acc.at[pl.ds(base + i * 40, 40)])
        return 0
    lax.fori_loop(0, ZPS // 40, zloop, 0)
    pltpu.sync_copy(zbuf.at[pl.ds(0, ZPS % 40)],
                    acc.at[pl.ds(base + (ZPS // 40) * 40, ZPS % 40)])
    plsc.subcore_barrier()

    _unpack_src(pidx, srcbuf, 0, 0)
    pltpu.async_copy(x_hbm.at[srcbuf.at[0]], gbuf.at[0], sem.at[0])

    def sloop(j, _):
        slot = lax.rem(j, 2)
        nslot = lax.rem(j + 1, 2)

        @pl.when(j < NCH32 - 1)
        def _():
            _unpack_src(pidx, srcbuf, j + 1, nslot)
            pltpu.async_copy(x_hbm.at[srcbuf.at[nslot]], gbuf.at[nslot],
                             sem.at[nslot])

        _unpack_dst(pidx, dstbuf, j)
        pltpu.make_async_copy(x_hbm.at[srcbuf.at[slot]], gbuf.at[slot],
                              sem.at[slot]).wait()
        pltpu.sync_copy(gbuf.at[slot], acc.at[dstbuf.at[0]], add=True)
        return 0
    lax.fori_loop(0, NCH32, sloop, 0)
    plsc.subcore_barrier()

    pltpu.sync_copy(acc.at[pl.ds(base, ZPS)], out_hbm.at[c, pl.ds(base, ZPS)])


def _sc_aggregate16(x1p, pidx32):
    return pl.kernel(
        _agg16_body,
        out_type=jax.ShapeDtypeStruct((NC, NP, LANES), jnp.float32),
        mesh=_MESH,
        scratch_types=[
            pltpu.VMEM((NCH32, CHUNK), jnp.int32),
            pltpu.VMEM((2, CHUNK), jnp.int32),
            pltpu.VMEM((1, CHUNK), jnp.int32),
            pltpu.VMEM((2, CHUNK, LANES), jnp.float32),
            pltpu.VMEM((40, LANES), jnp.float32),
            pltpu.VMEM_SHARED((NP, LANES), jnp.float32),
            pltpu.SemaphoreType.DMA((2,)),
        ],
    )(x1p, pidx32)


def _prep_body(deg_ref, x1_ref, ns_ref, nd_ref):
    ind = deg_ref[0, :, 0:1]
    outd = deg_ref[1, :, 0:1]
    h1 = ind
    h2 = (ind > 3.0).astype(jnp.float32)
    h3 = 3.0 / ind
    h4 = (ind > 4.0).astype(jnp.float32)
    ns = lax.rsqrt(jnp.where(outd > 0.0, outd, 1.0))
    nd = lax.rsqrt(jnp.where(ind > 0.0, ind, 1.0))
    ns_ref[...] = ns
    nd_ref[...] = nd
    x1_ref[...] = jnp.concatenate(
        [h1, h2, h3, h4, jnp.zeros((RB, LANES - 4), jnp.float32)],
        axis=1) * ns


def _tc_prep(deg):
    return pl.pallas_call(
        _prep_body,
        grid=(GRID,),
        in_specs=[
            pl.BlockSpec((NC, RB, 16), lambda r: (0, r, 0)),
        ],
        out_specs=[
            pl.BlockSpec((RB, LANES), lambda r: (r, 0)),
            pl.BlockSpec((RB, 1), lambda r: (r, 0)),
            pl.BlockSpec((RB, 1), lambda r: (r, 0)),
        ],
        out_shape=[
            jax.ShapeDtypeStruct((N, LANES), jnp.float32),
            jax.ShapeDtypeStruct((N, 1), jnp.float32),
            jax.ShapeDtypeStruct((N, 1), jnp.float32),
        ],
    )(deg)


def _mid1_body(agg_ref, nd_ref, ns_ref, w1_ref, b_ref, w_ref, t_ref):
    nd = nd_ref[...]
    ns = ns_ref[...]
    a = jnp.dot((agg_ref[0] + agg_ref[1])[:, :16], w1_ref[...],
                preferred_element_type=jnp.float32)
    y = jnp.maximum(a * nd + b_ref[...], 0.0) * ns
    for g in range(NPLANES):
        t_ref[g] = jnp.dot(y, w_ref[:, g * LANES:(g + 1) * LANES],
                           preferred_element_type=jnp.float32)


def _tc_mid1(agg16, nd, ns, w1p, b, w):
    return pl.pallas_call(
        _mid1_body,
        grid=(GRID,),
        in_specs=[
            pl.BlockSpec((NC, RB, LANES), lambda r: (0, r, 0)),
            pl.BlockSpec((RB, 1), lambda r: (r, 0)),
            pl.BlockSpec((RB, 1), lambda r: (r, 0)),
            pl.BlockSpec((16, HID), lambda r: (0, 0)),
            pl.BlockSpec((1, HID), lambda r: (0, 0)),
            pl.BlockSpec((HID, HID), lambda r: (0, 0)),
        ],
        out_specs=pl.BlockSpec((NPLANES, RB, LANES), lambda r: (0, r, 0)),
        out_shape=jax.ShapeDtypeStruct((NPLANES, N, LANES), jnp.float32),
    )(agg16, nd, ns, w1p, b, w)


def _mid_body(agg_ref, nd_ref, ns_ref, b_ref, w_ref, t_ref):
    nd = nd_ref[...]
    ns = ns_ref[...]
    ys = []
    for g in range(NPLANES):
        a = agg_ref[g]
        ys.append(jnp.maximum(a * nd + b_ref[0:1, g * LANES:(g + 1) * LANES],
                              0.0) * ns)
    y = jnp.concatenate(ys, axis=1)
    for g in range(NPLANES):
        t_ref[g] = jnp.dot(y, w_ref[:, g * LANES:(g + 1) * LANES],
                           preferred_element_type=jnp.float32)


def _tc_mid(agg, nd, ns, b, w):
    return pl.pallas_call(
        _mid_body,
        grid=(GRID,),
        in_specs=[
            pl.BlockSpec((NPLANES, RB, LANES), lambda r: (0, r, 0)),
            pl.BlockSpec((RB, 1), lambda r: (r, 0)),
            pl.BlockSpec((RB, 1), lambda r: (r, 0)),
            pl.BlockSpec((1, HID), lambda r: (0, 0)),
            pl.BlockSpec((HID, HID), lambda r: (0, 0)),
        ],
        out_specs=pl.BlockSpec((NPLANES, RB, LANES), lambda r: (0, r, 0)),
        out_shape=jax.ShapeDtypeStruct((NPLANES, N, LANES), jnp.float32),
    )(agg, nd, ns, b, w)


def _fin_body(agg_ref, nd_ref, b_ref, lw1_ref, lb1_ref, lw2_ref, lb2_ref,
              h_ref, pred_ref, ge_ref, acc_ref):
    r = pl.program_id(0)
    nd = nd_ref[...]
    hs = [jnp.maximum(agg_ref[g] * nd + b_ref[0:1, g * LANES:(g + 1) * LANES],
                      0.0) for g in range(NPLANES)]
    h = jnp.concatenate(hs, axis=1)
    h_ref[...] = h
    part = jnp.sum(h.reshape(RB // 8, 8, HID), axis=0)

    @pl.when(r == 0)
    def _():
        acc_ref[...] = part

    @pl.when(r > 0)
    def _():
        acc_ref[...] = acc_ref[...] + part

    @pl.when(r == GRID - 1)
    def _():
        sm = jnp.sum(acc_ref[...], axis=0, keepdims=True) / jnp.float32(N)
        ge_ref[...] = sm
        e = jnp.maximum(
            jnp.dot(sm, lw1_ref[...], preferred_element_type=jnp.float32)
            + lb1_ref[...], 0.0)
        p = (jnp.dot(e, lw2_ref[...], preferred_element_type=jnp.float32)
             + lb2_ref[...])
        pred_ref[...] = jax.nn.sigmoid(p)


def _tc_final(agg, nd, b, lw1, lb1, lw2, lb2):
    return pl.pallas_call(
        _fin_body,
        grid=(GRID,),
        in_specs=[
            pl.BlockSpec((NPLANES, RB, LANES), lambda r: (0, r, 0)),
            pl.BlockSpec((RB, 1), lambda r: (r, 0)),
            pl.BlockSpec((1, HID), lambda r: (0, 0)),
            pl.BlockSpec((HID, 256), lambda r: (0, 0)),
            pl.BlockSpec((1, 256), lambda r: (0, 0)),
            pl.BlockSpec((256, 1), lambda r: (0, 0)),
            pl.BlockSpec((1, 1), lambda r: (0, 0)),
        ],
        out_specs=[
            pl.BlockSpec((RB, HID), lambda r: (r, 0)),
            pl.BlockSpec((1, 1), lambda r: (0, 0)),
            pl.BlockSpec((1, HID), lambda r: (0, 0)),
        ],
        out_shape=[
            jax.ShapeDtypeStruct((N, HID), jnp.float32),
            jax.ShapeDtypeStruct((1, 1), jnp.float32),
            jax.ShapeDtypeStruct((1, HID), jnp.float32),
        ],
        scratch_shapes=[pltpu.VMEM((8, HID), jnp.float32)],
    )(agg, nd, b, lw1, lb1, lw2, lb2)


def kernel(edge_index, W1, b1, W2, b2, W3, b3, W4, b4, W5, b5,
           lw1, lb1, lw2, lb2):
    src = edge_index[0]
    dst = edge_index[1]
    padN = jnp.full((NS, EPAD - EPS), N, jnp.int32)
    pad0 = jnp.zeros((NS, EPAD - EPS), jnp.int32)
    dst_sub = jnp.concatenate([dst.reshape(NS, EPS), padN], axis=1)
    src_sub = jnp.concatenate([src.reshape(NS, EPS), pad0], axis=1)
    dstidx = dst_sub.reshape(NS, NCHUNKS, CHUNK)
    pidx = ((dst_sub << 16) | src_sub)[None] + \
        (jnp.arange(NPLANES, dtype=jnp.int32) * N)[:, None, None]
    pidx = pidx.reshape(NPLANES, NS, NCHUNKS, CHUNK)
    src_subN = jnp.concatenate([src.reshape(NS, EPS), padN], axis=1)
    degidx = jnp.stack([dstidx, src_subN.reshape(NS, NCHUNKS, CHUNK)])
    padN32 = jnp.full((NC * NS, EPAD32 - EPS32), N, jnp.int32)
    pad032 = jnp.zeros((NC * NS, EPAD32 - EPS32), jnp.int32)
    dst32 = jnp.concatenate([dst.reshape(NC * NS, EPS32), padN32], axis=1)
    src32 = jnp.concatenate([src.reshape(NC * NS, EPS32), pad032], axis=1)
    pidx32 = ((dst32 << 16) | src32).reshape(NC * NS, NCH32, CHUNK)

    deg = _sc_degrees(degidx)
    w1p = jnp.pad(W1, ((0, 12), (0, 0)))
    x1, ns, nd = _tc_prep(deg)
    agg16 = _sc_aggregate16(x1, pidx32)
    t = _tc_mid1(agg16, nd, ns, w1p, b1.reshape(1, HID), W2)
    agg = _sc_aggregate(t.reshape(NPLANES * N, LANES), pidx)
    for bprev, W in ((b2, W3), (b3, W4), (b4, W5)):
        t = _tc_mid(agg, nd, ns, bprev.reshape(1, HID), W)
        agg = _sc_aggregate(t.reshape(NPLANES * N, LANES), pidx)
    h_nodes, pred, graph_emb = _tc_final(
        agg, nd, b5.reshape(1, HID), lw1, lb1.reshape(1, 256),
        lw2, lb2.reshape(1, 1))
    return (pred, graph_emb, h_nodes)
